# baseline (speedup 1.0000x reference)
import functools

import jax
import jax.numpy as jnp
from jax import lax
from jax.experimental import pallas as pl
from jax.experimental.pallas import tpu as pltpu
from jax.experimental.pallas import tpu_sc as plsc

T = 4096
D = 1024
E = 16
HE = 256
HR = 256
K = 2
TB = 128
NBLK = T // TB
SLOTS = T * K
BT = 128
PMAX = SLOTS + E * BT
NTILES = PMAX // BT
NW = 32
SPW = SLOTS // NW
CHUNK = 32
NCH = SPW // CHUNK

_f32 = jnp.float32
_i32 = jnp.int32
_bf16 = jnp.bfloat16


def _router_body(x_ref, gw_ref, w3_ref, x16_ref, pos3_ref, meta_ref,
                 carry, e_all, r_all):
    i = pl.program_id(0)

    @pl.when(i == 0)
    def _():
        carry[...] = jnp.zeros_like(carry)

    @pl.when(i < NBLK)
    def _():
        xb = x_ref[...]
        x16_ref[...] = xb.astype(_bf16)
        logits = jnp.dot(xb, gw_ref[...], preferred_element_type=_f32)
        m = jnp.max(logits, axis=-1, keepdims=True)
        p = jnp.exp(logits - m)
        probs = p / jnp.sum(p, axis=-1, keepdims=True)

        i1 = jnp.argmax(probs, axis=-1)
        lanes = lax.broadcasted_iota(_i32, (TB, E), 1)
        oh1 = lanes == i1[:, None]
        v1 = jnp.max(probs, axis=-1)
        probs2 = jnp.where(oh1, -1.0, probs)
        i2 = jnp.argmax(probs2, axis=-1)
        oh2 = lanes == i2[:, None]
        v2 = jnp.max(probs2, axis=-1)

        w3_ref[0, 0] = v1
        w3_ref[0, 1] = v2

        e_slots = jnp.concatenate([i1, i2], axis=0).astype(_i32)
        e_all[pl.ds(i, 1), :] = e_slots.reshape(1, 2 * TB)

        onehot = jnp.concatenate([oh1, oh2], axis=0).astype(_f32)
        r0 = lax.broadcasted_iota(_i32, (2 * TB, 2 * TB), 0)
        r1 = lax.broadcasted_iota(_i32, (2 * TB, 2 * TB), 1)
        tri = (r0 > r1).astype(_f32)
        local = jnp.dot(tri, onehot, preferred_element_type=_f32)
        rank = jnp.sum((local + carry[...]) * onehot, axis=1)
        r_all[pl.ds(i, 1), :] = rank.reshape(1, 2 * TB)

        carry[...] = carry[...] + jnp.sum(onehot, axis=0, keepdims=True)

    @pl.when(i == NBLK)
    def _():
        c = carry[...]
        pc = jnp.ceil(c / BT) * BT
        r0 = lax.broadcasted_iota(_i32, (E, E), 0)
        r1 = lax.broadcasted_iota(_i32, (E, E), 1)
        tri_u = (r0 < r1).astype(_f32)
        off = jnp.dot(pc, tri_u, preferred_element_type=_f32)

        e = e_all[...]
        acc = jnp.zeros((NBLK, 2 * TB), _f32)
        for ee in range(E):
            acc = acc + jnp.where(e == ee, off[0:1, ee:ee + 1], 0.0)
        pos3_ref[...] = (acc + r_all[...]).astype(_i32).reshape(NBLK, 1, 2 * TB)

        cum = off + pc
        ts = lax.broadcasted_iota(_i32, (TB, E), 0).astype(_f32) * BT
        te = jnp.sum((ts >= cum).astype(_f32), axis=1)
        te = jnp.minimum(te, float(E - 1))
        total = jnp.sum(pc)
        lane = lax.broadcasted_iota(_i32, (1, TB), 1)
        vec = jnp.where(lane < NTILES, te.reshape(1, TB),
                        jnp.where(lane == NTILES, total, 0.0))
        meta_ref[...] = vec.astype(_i32)


def _router(x, gate_W):
    clamp = NBLK - 1
    return pl.pallas_call(
        _router_body,
        grid=(NBLK + 1,),
        in_specs=[
            pl.BlockSpec((TB, D), lambda i: (jnp.minimum(i, clamp), 0)),
            pl.BlockSpec((D, E), lambda i: (0, 0)),
        ],
        out_specs=[
            pl.BlockSpec((1, 2, TB), lambda i: (jnp.minimum(i, clamp), 0, 0)),
            pl.BlockSpec((TB, D), lambda i: (jnp.minimum(i, clamp), 0)),
            pl.BlockSpec((NBLK, 1, 2 * TB), lambda i: (0, 0, 0)),
            pl.BlockSpec((1, TB), lambda i: (0, 0)),
        ],
        out_shape=[
            jax.ShapeDtypeStruct((NBLK, 2, TB), _f32),
            jax.ShapeDtypeStruct((T, D), _bf16),
            jax.ShapeDtypeStruct((NBLK, 1, 2 * TB), _i32),
            jax.ShapeDtypeStruct((1, TB), _i32),
        ],
        scratch_shapes=[
            pltpu.VMEM((1, E), _f32),
            pltpu.VMEM((NBLK, 2 * TB), _i32),
            pltpu.VMEM((NBLK, 2 * TB), _f32),
        ],
        compiler_params=pltpu.CompilerParams(
            dimension_semantics=("arbitrary",)),
    )(x, gate_W)


def _sc_permute(x, pos3d, tok3d):
    mesh = plsc.VectorSubcoreMesh(core_axis_name="c", subcore_axis_name="s")

    @functools.partial(
        pl.kernel, mesh=mesh,
        out_type=jax.ShapeDtypeStruct((PMAX, D), _f32),
        scratch_types=[
            pltpu.VMEM((NCH, CHUNK), _i32),
            pltpu.VMEM((NCH, CHUNK), _i32),
            pltpu.VMEM((CHUNK, D), _f32),
            pltpu.VMEM((CHUNK, D), _f32),
            pltpu.SemaphoreType.DMA,
            pltpu.SemaphoreType.DMA,
            pltpu.SemaphoreType.DMA,
            pltpu.SemaphoreType.DMA,
        ],
    )
    def k(x_hbm, pos_hbm, tok_hbm, out_hbm,
          tok_v, pos_v, rows0, rows1, g0, g1, s0, s1):
        wid = lax.axis_index("s") * 2 + lax.axis_index("c")
        pltpu.sync_copy(tok_hbm.at[wid], tok_v)
        pltpu.sync_copy(pos_hbm.at[wid], pos_v)
        rows = (rows0, rows1)
        gsem = (g0, g1)
        ssem = (s0, s1)
        gd = [None] * NCH
        sd = [None] * NCH
        gd[0] = pltpu.async_copy(x_hbm.at[tok_v.at[0]], rows[0], gsem[0])
        for ch in range(NCH):
            b = ch % 2
            if ch > 0:
                sd[ch - 1].wait()
            if ch + 1 < NCH:
                gd[ch + 1] = pltpu.async_copy(
                    x_hbm.at[tok_v.at[ch + 1]], rows[1 - b], gsem[1 - b])
            gd[ch].wait()
            sd[ch] = pltpu.async_copy(rows[b], out_hbm.at[pos_v.at[ch]],
                                      ssem[b])
        sd[NCH - 1].wait()

    return k(x, pos3d, tok3d)


def _sc_gather(y, pos3d):
    mesh = plsc.VectorSubcoreMesh(core_axis_name="c", subcore_axis_name="s")

    @functools.partial(
        pl.kernel, mesh=mesh,
        out_type=jax.ShapeDtypeStruct((SLOTS, D), _f32),
        scratch_types=[
            pltpu.VMEM((NCH, CHUNK), _i32),
            pltpu.VMEM((CHUNK, D), _f32),
            pltpu.VMEM((CHUNK, D), _f32),
            pltpu.SemaphoreType.DMA,
            pltpu.SemaphoreType.DMA,
            pltpu.SemaphoreType.DMA,
            pltpu.SemaphoreType.DMA,
        ],
    )
    def k(y_hbm, pos_hbm, out_hbm, pos_v, rows0, rows1, g0, g1, s0, s1):
        wid = lax.axis_index("s") * 2 + lax.axis_index("c")
        base = wid * SPW
        pltpu.sync_copy(pos_hbm.at[wid], pos_v)
        rows = (rows0, rows1)
        gsem = (g0, g1)
        ssem = (s0, s1)
        gd = [None] * NCH
        sd = [None] * NCH
        gd[0] = pltpu.async_copy(y_hbm.at[pos_v.at[0]], rows[0], gsem[0])
        for ch in range(NCH):
            b = ch % 2
            if ch > 0:
                sd[ch - 1].wait()
            if ch + 1 < NCH:
                gd[ch + 1] = pltpu.async_copy(
                    y_hbm.at[pos_v.at[ch + 1]], rows[1 - b], gsem[1 - b])
            gd[ch].wait()
            sd[ch] = pltpu.async_copy(
                rows[b], out_hbm.at[pl.ds(base + ch * CHUNK, CHUNK)], ssem[b])
        sd[NCH - 1].wait()

    return k(y, pos3d)


def _grouped_body(meta_ref, xs_ref, wg_ref, wu_ref, wd_ref,
                  bg_ref, bu_ref, bd_ref, y_ref):
    i = pl.program_id(0)

    @pl.when(i * BT < meta_ref[NTILES])
    def _():
        xt = xs_ref[...].astype(_bf16)
        g = jnp.dot(xt, wg_ref[0], preferred_element_type=_f32) + bg_ref[0, 0]
        u = jnp.dot(xt, wu_ref[0], preferred_element_type=_f32) + bu_ref[0, 0]
        h = (jax.nn.silu(g) * u).astype(_bf16)
        y = jnp.dot(h, wd_ref[0], preferred_element_type=_f32)
        y_ref[...] = y + bd_ref[0, 0]


def _grouped(meta, xs, wg, wu, wd, bg, bu, bd):
    grid_spec = pltpu.PrefetchScalarGridSpec(
        num_scalar_prefetch=1,
        grid=(NTILES,),
        in_specs=[
            pl.BlockSpec((BT, D), lambda i, m: (i, 0)),
            pl.BlockSpec((1, D, HE), lambda i, m: (m[i], 0, 0)),
            pl.BlockSpec((1, D, HE), lambda i, m: (m[i], 0, 0)),
            pl.BlockSpec((1, HE, D), lambda i, m: (m[i], 0, 0)),
            pl.BlockSpec((1, 1, HE), lambda i, m: (m[i], 0, 0)),
            pl.BlockSpec((1, 1, HE), lambda i, m: (m[i], 0, 0)),
            pl.BlockSpec((1, 1, D), lambda i, m: (m[i], 0, 0)),
        ],
        out_specs=pl.BlockSpec((BT, D), lambda i, m: (i, 0)),
    )
    return pl.pallas_call(
        _grouped_body,
        grid_spec=grid_spec,
        out_shape=jax.ShapeDtypeStruct((PMAX, D), _f32),
        compiler_params=pltpu.CompilerParams(
            dimension_semantics=("arbitrary",)),
    )(meta, xs, wg, wu, wd, bg, bu, bd)


def _combine_body(x_ref, z_ref, w3_ref, wrg_ref, wru_ref, wrd_ref,
                  brg_ref, bru_ref, brd_ref, out_ref):
    zb = z_ref[...]
    w1 = w3_ref[0, 0]
    w2 = w3_ref[0, 1]
    moe = w1[:, None] * zb[:TB] + w2[:, None] * zb[TB:]
    xt = x_ref[...].astype(_f32)
    g = jnp.dot(xt, wrg_ref[...], preferred_element_type=_f32) + brg_ref[...]
    u = jnp.dot(xt, wru_ref[...], preferred_element_type=_f32) + bru_ref[...]
    h = jax.nn.silu(g) * u
    res = jnp.dot(h, wrd_ref[...], preferred_element_type=_f32) + brd_ref[...]
    out_ref[...] = moe + res


def _combine(x16, z, w3, wrg, wru, wrd, brg, bru, brd):
    return pl.pallas_call(
        _combine_body,
        grid=(NBLK,),
        in_specs=[
            pl.BlockSpec((TB, D), lambda i: (i, 0)),
            pl.BlockSpec((2 * TB, D), lambda i: (i, 0)),
            pl.BlockSpec((1, 2, TB), lambda i: (i, 0, 0)),
            pl.BlockSpec((D, HR), lambda i: (0, 0)),
            pl.BlockSpec((D, HR), lambda i: (0, 0)),
            pl.BlockSpec((HR, D), lambda i: (0, 0)),
            pl.BlockSpec((1, HR), lambda i: (0, 0)),
            pl.BlockSpec((1, HR), lambda i: (0, 0)),
            pl.BlockSpec((1, D), lambda i: (0, 0)),
        ],
        out_specs=pl.BlockSpec((TB, D), lambda i: (i, 0)),
        out_shape=jax.ShapeDtypeStruct((T, D), _f32),
        compiler_params=pltpu.CompilerParams(
            dimension_semantics=("arbitrary",)),
    )(x16, z, w3, wrg, wru, wrd, brg, bru, brd)


def kernel(x, gate_W, W_gate, W_up, W_down, b_gate, b_up, b_down,
           Wr_gate, Wr_up, Wr_down, br_gate, br_up, br_down):
    w3, x16, pos3, meta = _router(x, gate_W)
    pos3d = pos3.reshape(NW, NCH, CHUNK)
    meta = meta.reshape(TB)
    sidx = jnp.arange(SLOTS, dtype=_i32)
    tok3d = ((sidx // (2 * TB)) * TB + sidx % TB).reshape(NW, NCH, CHUNK)

    xs = _sc_permute(x, pos3d, tok3d)
    y = _grouped(meta, xs, W_gate.astype(_bf16), W_up.astype(_bf16),
                 W_down.astype(_bf16),
                 b_gate.reshape(E, 1, HE), b_up.reshape(E, 1, HE),
                 b_down.reshape(E, 1, D))
    z = _sc_gather(y, pos3d)

    return _combine(x16, z, w3, Wr_gate, Wr_up, Wr_down,
                    br_gate.reshape(1, HR), br_up.reshape(1, HR),
                    br_down.reshape(1, D))


# speedup vs baseline: 1.0947x; 1.0947x over previous
import functools

import jax
import jax.numpy as jnp
from jax import lax
from jax.experimental import pallas as pl
from jax.experimental.pallas import tpu as pltpu
from jax.experimental.pallas import tpu_sc as plsc

T = 4096
D = 1024
E = 16
HE = 256
HR = 256
K = 2
TB = 128
NBLK = T // TB
SLOTS = T * K
BT = 128
PMAX = SLOTS + E * BT
NTILES = PMAX // BT
NW = 32
SPW = SLOTS // NW
CHUNK = 32
NCH = SPW // CHUNK

_f32 = jnp.float32
_i32 = jnp.int32
_bf16 = jnp.bfloat16


def _router_body(x_ref, gw_ref, w3_ref, x16_ref, pos3_ref, meta_ref,
                 carry, e_all, r_all, triu):
    i = pl.program_id(0)

    @pl.when(i == 0)
    def _():
        carry[...] = jnp.zeros_like(carry)
        t0 = lax.broadcasted_iota(_i32, (2 * TB, 2 * TB), 0)
        t1 = lax.broadcasted_iota(_i32, (2 * TB, 2 * TB), 1)
        triu[...] = (t0 < t1).astype(_f32)

    @pl.when(i < NBLK)
    def _():
        xb = x_ref[...]
        x16_ref[...] = xb.astype(_bf16)
        logits = jnp.dot(xb, gw_ref[...], preferred_element_type=_f32)
        lt = logits.T
        m = jnp.max(lt, axis=0, keepdims=True)
        p = jnp.exp(lt - m)
        probs = p / jnp.sum(p, axis=0, keepdims=True)

        i1 = jnp.argmax(probs, axis=0)
        subl = lax.broadcasted_iota(_i32, (E, TB), 0)
        oh1 = subl == i1[None, :]
        v1 = jnp.max(probs, axis=0)
        probs2 = jnp.where(oh1, -1.0, probs)
        i2 = jnp.argmax(probs2, axis=0)
        oh2 = subl == i2[None, :]
        v2 = jnp.max(probs2, axis=0)

        w3_ref[0, 0] = v1
        w3_ref[0, 1] = v2
        e_slots = jnp.concatenate([i1, i2], axis=0).astype(_i32)
        e_all[pl.ds(i, 1), :] = e_slots.reshape(1, 2 * TB)

        ohT = jnp.concatenate([oh1, oh2], axis=1).astype(_f32)
        local = jnp.dot(ohT, triu[...], preferred_element_type=_f32)
        rank = jnp.sum((local + carry[...]) * ohT, axis=0)
        r_all[pl.ds(i, 1), :] = rank.reshape(1, 2 * TB)

        carry[...] = carry[...] + jnp.sum(ohT, axis=1, keepdims=True)

    @pl.when(i == NBLK)
    def _():
        c = carry[...]
        pc = jnp.ceil(c / BT) * BT
        r0 = lax.broadcasted_iota(_i32, (E, E), 0)
        r1 = lax.broadcasted_iota(_i32, (E, E), 1)
        tri_l = (r1 < r0).astype(_f32)
        off = jnp.dot(tri_l, pc, preferred_element_type=_f32)

        e = e_all[...]
        acc = jnp.zeros((NBLK, 2 * TB), _f32)
        for ee in range(E):
            acc = acc + jnp.where(e == ee, off[ee:ee + 1, 0:1], 0.0)
        pos3_ref[...] = (acc + r_all[...]).astype(_i32).reshape(NBLK, 1, 2 * TB)

        cum = off + pc
        ts = lax.broadcasted_iota(_i32, (E, TB), 1).astype(_f32) * BT
        te = jnp.sum((ts >= cum).astype(_f32), axis=0)
        te = jnp.minimum(te, float(E - 1))
        total = jnp.sum(pc)
        lane = lax.broadcasted_iota(_i32, (1, TB), 1)
        vec = jnp.where(lane < NTILES, te.reshape(1, TB),
                        jnp.where(lane == NTILES, total, 0.0))
        meta_ref[...] = vec.astype(_i32)


def _router(x, gate_W):
    clamp = NBLK - 1
    return pl.pallas_call(
        _router_body,
        grid=(NBLK + 1,),
        in_specs=[
            pl.BlockSpec((TB, D), lambda i: (jnp.minimum(i, clamp), 0)),
            pl.BlockSpec((D, E), lambda i: (0, 0)),
        ],
        out_specs=[
            pl.BlockSpec((1, 2, TB), lambda i: (jnp.minimum(i, clamp), 0, 0)),
            pl.BlockSpec((TB, D), lambda i: (jnp.minimum(i, clamp), 0)),
            pl.BlockSpec((NBLK, 1, 2 * TB), lambda i: (0, 0, 0)),
            pl.BlockSpec((1, TB), lambda i: (0, 0)),
        ],
        out_shape=[
            jax.ShapeDtypeStruct((NBLK, 2, TB), _f32),
            jax.ShapeDtypeStruct((T, D), _bf16),
            jax.ShapeDtypeStruct((NBLK, 1, 2 * TB), _i32),
            jax.ShapeDtypeStruct((1, TB), _i32),
        ],
        scratch_shapes=[
            pltpu.VMEM((E, 1), _f32),
            pltpu.VMEM((NBLK, 2 * TB), _i32),
            pltpu.VMEM((NBLK, 2 * TB), _f32),
            pltpu.VMEM((2 * TB, 2 * TB), _f32),
        ],
        compiler_params=pltpu.CompilerParams(
            dimension_semantics=("arbitrary",)),
    )(x, gate_W)


def _sc_permute(x, pos3d, tok3d):
    mesh = plsc.VectorSubcoreMesh(core_axis_name="c", subcore_axis_name="s")

    @functools.partial(
        pl.kernel, mesh=mesh,
        out_type=jax.ShapeDtypeStruct((PMAX, D), _f32),
        scratch_types=[
            pltpu.VMEM((NCH, CHUNK), _i32),
            pltpu.VMEM((NCH, CHUNK), _i32),
            pltpu.VMEM((CHUNK, D), _f32),
            pltpu.VMEM((CHUNK, D), _f32),
            pltpu.SemaphoreType.DMA,
            pltpu.SemaphoreType.DMA,
            pltpu.SemaphoreType.DMA,
            pltpu.SemaphoreType.DMA,
        ],
    )
    def k(x_hbm, pos_hbm, tok_hbm, out_hbm,
          tok_v, pos_v, rows0, rows1, g0, g1, s0, s1):
        wid = lax.axis_index("s") * 2 + lax.axis_index("c")
        pltpu.sync_copy(tok_hbm.at[wid], tok_v)
        pltpu.sync_copy(pos_hbm.at[wid], pos_v)
        rows = (rows0, rows1)
        gsem = (g0, g1)
        ssem = (s0, s1)
        gd = [None] * NCH
        sd = [None] * NCH
        gd[0] = pltpu.async_copy(x_hbm.at[tok_v.at[0]], rows[0], gsem[0])
        for ch in range(NCH):
            b = ch % 2
            if ch > 0:
                sd[ch - 1].wait()
            if ch + 1 < NCH:
                gd[ch + 1] = pltpu.async_copy(
                    x_hbm.at[tok_v.at[ch + 1]], rows[1 - b], gsem[1 - b])
            gd[ch].wait()
            sd[ch] = pltpu.async_copy(rows[b], out_hbm.at[pos_v.at[ch]],
                                      ssem[b])
        sd[NCH - 1].wait()

    return k(x, pos3d, tok3d)


def _sc_gather(y, pos3d):
    mesh = plsc.VectorSubcoreMesh(core_axis_name="c", subcore_axis_name="s")

    @functools.partial(
        pl.kernel, mesh=mesh,
        out_type=jax.ShapeDtypeStruct((SLOTS, D), _f32),
        scratch_types=[
            pltpu.VMEM((NCH, CHUNK), _i32),
            pltpu.VMEM((CHUNK, D), _f32),
            pltpu.VMEM((CHUNK, D), _f32),
            pltpu.SemaphoreType.DMA,
            pltpu.SemaphoreType.DMA,
            pltpu.SemaphoreType.DMA,
            pltpu.SemaphoreType.DMA,
        ],
    )
    def k(y_hbm, pos_hbm, out_hbm, pos_v, rows0, rows1, g0, g1, s0, s1):
        wid = lax.axis_index("s") * 2 + lax.axis_index("c")
        base = wid * SPW
        pltpu.sync_copy(pos_hbm.at[wid], pos_v)
        rows = (rows0, rows1)
        gsem = (g0, g1)
        ssem = (s0, s1)
        gd = [None] * NCH
        sd = [None] * NCH
        gd[0] = pltpu.async_copy(y_hbm.at[pos_v.at[0]], rows[0], gsem[0])
        for ch in range(NCH):
            b = ch % 2
            if ch > 0:
                sd[ch - 1].wait()
            if ch + 1 < NCH:
                gd[ch + 1] = pltpu.async_copy(
                    y_hbm.at[pos_v.at[ch + 1]], rows[1 - b], gsem[1 - b])
            gd[ch].wait()
            sd[ch] = pltpu.async_copy(
                rows[b], out_hbm.at[pl.ds(base + ch * CHUNK, CHUNK)], ssem[b])
        sd[NCH - 1].wait()

    return k(y, pos3d)


def _grouped_body(meta_ref, xs_ref, wg_ref, wu_ref, wd_ref,
                  bg_ref, bu_ref, bd_ref, y_ref):
    i = pl.program_id(0)

    @pl.when(i * BT < meta_ref[NTILES])
    def _():
        xt = xs_ref[...]
        g = jnp.dot(xt, wg_ref[0], preferred_element_type=_f32) + bg_ref[0, 0]
        u = jnp.dot(xt, wu_ref[0], preferred_element_type=_f32) + bu_ref[0, 0]
        h = jax.nn.silu(g) * u
        y = jnp.dot(h, wd_ref[0], preferred_element_type=_f32)
        y_ref[...] = y + bd_ref[0, 0]


def _grouped(meta, xs, wg, wu, wd, bg, bu, bd):
    grid_spec = pltpu.PrefetchScalarGridSpec(
        num_scalar_prefetch=1,
        grid=(NTILES,),
        in_specs=[
            pl.BlockSpec((BT, D), lambda i, m: (i, 0)),
            pl.BlockSpec((1, D, HE), lambda i, m: (m[i], 0, 0)),
            pl.BlockSpec((1, D, HE), lambda i, m: (m[i], 0, 0)),
            pl.BlockSpec((1, HE, D), lambda i, m: (m[i], 0, 0)),
            pl.BlockSpec((1, 1, HE), lambda i, m: (m[i], 0, 0)),
            pl.BlockSpec((1, 1, HE), lambda i, m: (m[i], 0, 0)),
            pl.BlockSpec((1, 1, D), lambda i, m: (m[i], 0, 0)),
        ],
        out_specs=pl.BlockSpec((BT, D), lambda i, m: (i, 0)),
    )
    return pl.pallas_call(
        _grouped_body,
        grid_spec=grid_spec,
        out_shape=jax.ShapeDtypeStruct((PMAX, D), _f32),
        compiler_params=pltpu.CompilerParams(
            dimension_semantics=("arbitrary",)),
    )(meta, xs, wg, wu, wd, bg, bu, bd)


def _combine_body(x_ref, z_ref, w3_ref, wrg_ref, wru_ref, wrd_ref,
                  brg_ref, bru_ref, brd_ref, out_ref):
    zb = z_ref[...]
    w1 = w3_ref[0, 0]
    w2 = w3_ref[0, 1]
    moe = w1[:, None] * zb[:TB] + w2[:, None] * zb[TB:]
    xt = x_ref[...].astype(_f32)
    g = jnp.dot(xt, wrg_ref[...], preferred_element_type=_f32) + brg_ref[...]
    u = jnp.dot(xt, wru_ref[...], preferred_element_type=_f32) + bru_ref[...]
    h = jax.nn.silu(g) * u
    res = jnp.dot(h, wrd_ref[...], preferred_element_type=_f32) + brd_ref[...]
    out_ref[...] = moe + res


def _combine(x16, z, w3, wrg, wru, wrd, brg, bru, brd):
    return pl.pallas_call(
        _combine_body,
        grid=(NBLK,),
        in_specs=[
            pl.BlockSpec((TB, D), lambda i: (i, 0)),
            pl.BlockSpec((2 * TB, D), lambda i: (i, 0)),
            pl.BlockSpec((1, 2, TB), lambda i: (i, 0, 0)),
            pl.BlockSpec((D, HR), lambda i: (0, 0)),
            pl.BlockSpec((D, HR), lambda i: (0, 0)),
            pl.BlockSpec((HR, D), lambda i: (0, 0)),
            pl.BlockSpec((1, HR), lambda i: (0, 0)),
            pl.BlockSpec((1, HR), lambda i: (0, 0)),
            pl.BlockSpec((1, D), lambda i: (0, 0)),
        ],
        out_specs=pl.BlockSpec((TB, D), lambda i: (i, 0)),
        out_shape=jax.ShapeDtypeStruct((T, D), _f32),
        compiler_params=pltpu.CompilerParams(
            dimension_semantics=("arbitrary",)),
    )(x16, z, w3, wrg, wru, wrd, brg, bru, brd)


def kernel(x, gate_W, W_gate, W_up, W_down, b_gate, b_up, b_down,
           Wr_gate, Wr_up, Wr_down, br_gate, br_up, br_down):
    w3, x16, pos3, meta = _router(x, gate_W)
    pos3d = pos3.reshape(NW, NCH, CHUNK)
    meta = meta.reshape(TB)
    sidx = jnp.arange(SLOTS, dtype=_i32)
    tok3d = ((sidx // (2 * TB)) * TB + sidx % TB).reshape(NW, NCH, CHUNK)

    xs = _sc_permute(x, pos3d, tok3d)
    y = _grouped(meta, xs, W_gate, W_up, W_down,
                 b_gate.reshape(E, 1, HE), b_up.reshape(E, 1, HE),
                 b_down.reshape(E, 1, D))
    z = _sc_gather(y, pos3d)

    return _combine(x16, z, w3, Wr_gate, Wr_up, Wr_down,
                    br_gate.reshape(1, HR), br_up.reshape(1, HR),
                    br_down.reshape(1, D))


# speedup vs baseline: 1.1077x; 1.0119x over previous
import functools

import jax
import jax.numpy as jnp
from jax import lax
from jax.experimental import pallas as pl
from jax.experimental.pallas import tpu as pltpu
from jax.experimental.pallas import tpu_sc as plsc

T = 4096
D = 1024
E = 16
HE = 256
HR = 256
K = 2
TB = 128
NBLK = T // TB
SLOTS = T * K
BT = 128
PMAX = SLOTS + E * BT
NTILES = PMAX // BT
NW = 32
SPW = SLOTS // NW
CHUNK = 32
NCH = SPW // CHUNK

_f32 = jnp.float32
_i32 = jnp.int32
_bf16 = jnp.bfloat16


def _router_body(x_ref, gw_ref, w3_ref, x16_ref, pos3_ref, meta_ref,
                 carry, e_all, r_all, triu):
    i = pl.program_id(0)

    @pl.when(i == 0)
    def _():
        carry[...] = jnp.zeros_like(carry)
        t0 = lax.broadcasted_iota(_i32, (2 * TB, 2 * TB), 0)
        t1 = lax.broadcasted_iota(_i32, (2 * TB, 2 * TB), 1)
        triu[...] = (t0 < t1).astype(_f32)

    @pl.when(i < NBLK)
    def _():
        xb = x_ref[...]
        x16_ref[...] = xb.astype(_bf16)
        logits = jnp.dot(xb, gw_ref[...], preferred_element_type=_f32)
        lt = logits.T
        m = jnp.max(lt, axis=0, keepdims=True)
        p = jnp.exp(lt - m)
        probs = p / jnp.sum(p, axis=0, keepdims=True)

        i1 = jnp.argmax(probs, axis=0)
        subl = lax.broadcasted_iota(_i32, (E, TB), 0)
        oh1 = subl == i1[None, :]
        v1 = jnp.max(probs, axis=0)
        probs2 = jnp.where(oh1, -1.0, probs)
        i2 = jnp.argmax(probs2, axis=0)
        oh2 = subl == i2[None, :]
        v2 = jnp.max(probs2, axis=0)

        w3_ref[0, 0] = v1
        w3_ref[0, 1] = v2
        e_slots = jnp.concatenate([i1, i2], axis=0).astype(_i32)
        e_all[pl.ds(i, 1), :] = e_slots.reshape(1, 2 * TB)

        ohT = jnp.concatenate([oh1, oh2], axis=1).astype(_f32)
        local = jnp.dot(ohT, triu[...], preferred_element_type=_f32)
        rank = jnp.sum((local + carry[...]) * ohT, axis=0)
        r_all[pl.ds(i, 1), :] = rank.reshape(1, 2 * TB)

        carry[...] = carry[...] + jnp.sum(ohT, axis=1, keepdims=True)

    @pl.when(i == NBLK)
    def _():
        c = carry[...]
        pc = jnp.ceil(c / BT) * BT
        r0 = lax.broadcasted_iota(_i32, (E, E), 0)
        r1 = lax.broadcasted_iota(_i32, (E, E), 1)
        tri_l = (r1 < r0).astype(_f32)
        off = jnp.dot(tri_l, pc, preferred_element_type=_f32)

        e = e_all[...]
        acc = jnp.zeros((NBLK, 2 * TB), _f32)
        for ee in range(E):
            acc = acc + jnp.where(e == ee, off[ee:ee + 1, 0:1], 0.0)
        pos3_ref[...] = (acc + r_all[...]).astype(_i32).reshape(NBLK, 1, 2 * TB)

        cum = off + pc
        ts = lax.broadcasted_iota(_i32, (E, TB), 1).astype(_f32) * BT
        te = jnp.sum((ts >= cum).astype(_f32), axis=0)
        te = jnp.minimum(te, float(E - 1))
        total = jnp.sum(pc)
        lane = lax.broadcasted_iota(_i32, (1, TB), 1)
        vec = jnp.where(lane < NTILES, te.reshape(1, TB),
                        jnp.where(lane == NTILES, total, 0.0))
        meta_ref[...] = vec.astype(_i32)


def _router(x, gate_W):
    clamp = NBLK - 1
    return pl.pallas_call(
        _router_body,
        grid=(NBLK + 1,),
        in_specs=[
            pl.BlockSpec((TB, D), lambda i: (jnp.minimum(i, clamp), 0)),
            pl.BlockSpec((D, E), lambda i: (0, 0)),
        ],
        out_specs=[
            pl.BlockSpec((1, 2, TB), lambda i: (jnp.minimum(i, clamp), 0, 0)),
            pl.BlockSpec((TB, D), lambda i: (jnp.minimum(i, clamp), 0)),
            pl.BlockSpec((NBLK, 1, 2 * TB), lambda i: (0, 0, 0)),
            pl.BlockSpec((1, TB), lambda i: (0, 0)),
        ],
        out_shape=[
            jax.ShapeDtypeStruct((NBLK, 2, TB), _f32),
            jax.ShapeDtypeStruct((T, D), _bf16),
            jax.ShapeDtypeStruct((NBLK, 1, 2 * TB), _i32),
            jax.ShapeDtypeStruct((1, TB), _i32),
        ],
        scratch_shapes=[
            pltpu.VMEM((E, 1), _f32),
            pltpu.VMEM((NBLK, 2 * TB), _i32),
            pltpu.VMEM((NBLK, 2 * TB), _f32),
            pltpu.VMEM((2 * TB, 2 * TB), _f32),
        ],
        compiler_params=pltpu.CompilerParams(
            dimension_semantics=("arbitrary",)),
    )(x, gate_W)


def _sc_permute(x, pos3d, tok3d):
    mesh = plsc.VectorSubcoreMesh(core_axis_name="c", subcore_axis_name="s")

    @functools.partial(
        pl.kernel, mesh=mesh,
        out_type=jax.ShapeDtypeStruct((PMAX, D), _f32),
        scratch_types=[
            pltpu.VMEM((NCH, CHUNK), _i32),
            pltpu.VMEM((NCH, CHUNK), _i32),
            pltpu.VMEM((CHUNK, D), _f32),
            pltpu.VMEM((CHUNK, D), _f32),
            pltpu.SemaphoreType.DMA,
            pltpu.SemaphoreType.DMA,
            pltpu.SemaphoreType.DMA,
            pltpu.SemaphoreType.DMA,
        ],
    )
    def k(x_hbm, pos_hbm, tok_hbm, out_hbm,
          tok_v, pos_v, rows0, rows1, g0, g1, s0, s1):
        wid = lax.axis_index("s") * 2 + lax.axis_index("c")
        pltpu.sync_copy(tok_hbm.at[wid], tok_v)
        pltpu.sync_copy(pos_hbm.at[wid], pos_v)
        rows = (rows0, rows1)
        gsem = (g0, g1)
        ssem = (s0, s1)
        gd = [None] * NCH
        sd = [None] * NCH
        gd[0] = pltpu.async_copy(x_hbm.at[tok_v.at[0]], rows[0], gsem[0])
        for ch in range(NCH):
            b = ch % 2
            if ch > 0:
                sd[ch - 1].wait()
            if ch + 1 < NCH:
                gd[ch + 1] = pltpu.async_copy(
                    x_hbm.at[tok_v.at[ch + 1]], rows[1 - b], gsem[1 - b])
            gd[ch].wait()
            sd[ch] = pltpu.async_copy(rows[b], out_hbm.at[pos_v.at[ch]],
                                      ssem[b])
        sd[NCH - 1].wait()

    return k(x, pos3d, tok3d)


def _sc_gather(y, pos3d):
    mesh = plsc.VectorSubcoreMesh(core_axis_name="c", subcore_axis_name="s")

    @functools.partial(
        pl.kernel, mesh=mesh,
        out_type=jax.ShapeDtypeStruct((SLOTS, D), _f32),
        scratch_types=[
            pltpu.VMEM((NCH, CHUNK), _i32),
            pltpu.VMEM((CHUNK, D), _f32),
            pltpu.VMEM((CHUNK, D), _f32),
            pltpu.SemaphoreType.DMA,
            pltpu.SemaphoreType.DMA,
            pltpu.SemaphoreType.DMA,
            pltpu.SemaphoreType.DMA,
        ],
    )
    def k(y_hbm, pos_hbm, out_hbm, pos_v, rows0, rows1, g0, g1, s0, s1):
        wid = lax.axis_index("s") * 2 + lax.axis_index("c")
        base = wid * SPW
        pltpu.sync_copy(pos_hbm.at[wid], pos_v)
        rows = (rows0, rows1)
        gsem = (g0, g1)
        ssem = (s0, s1)
        gd = [None] * NCH
        sd = [None] * NCH
        gd[0] = pltpu.async_copy(y_hbm.at[pos_v.at[0]], rows[0], gsem[0])
        for ch in range(NCH):
            b = ch % 2
            if ch > 0:
                sd[ch - 1].wait()
            if ch + 1 < NCH:
                gd[ch + 1] = pltpu.async_copy(
                    y_hbm.at[pos_v.at[ch + 1]], rows[1 - b], gsem[1 - b])
            gd[ch].wait()
            sd[ch] = pltpu.async_copy(
                rows[b], out_hbm.at[pl.ds(base + ch * CHUNK, CHUNK)], ssem[b])
        sd[NCH - 1].wait()

    return k(y, pos3d)


def _grouped_body(meta_ref, xs_ref, wg_ref, wu_ref, wd_ref,
                  bg_ref, bu_ref, bd_ref, y_ref):
    i = pl.program_id(0)

    @pl.when(i * BT < meta_ref[NTILES])
    def _():
        xt = xs_ref[...]
        g = jnp.dot(xt, wg_ref[0], preferred_element_type=_f32) + bg_ref[0, 0]
        u = jnp.dot(xt, wu_ref[0], preferred_element_type=_f32) + bu_ref[0, 0]
        h = jax.nn.silu(g) * u
        y = jnp.dot(h, wd_ref[0], preferred_element_type=_f32)
        y_ref[...] = y + bd_ref[0, 0]


def _grouped(meta, xs, wg, wu, wd, bg, bu, bd):
    def _act(i, m):
        return jnp.minimum(i, m[NTILES] // BT - 1)

    grid_spec = pltpu.PrefetchScalarGridSpec(
        num_scalar_prefetch=1,
        grid=(NTILES,),
        in_specs=[
            pl.BlockSpec((BT, D), lambda i, m: (_act(i, m), 0)),
            pl.BlockSpec((1, D, HE), lambda i, m: (m[_act(i, m)], 0, 0)),
            pl.BlockSpec((1, D, HE), lambda i, m: (m[_act(i, m)], 0, 0)),
            pl.BlockSpec((1, HE, D), lambda i, m: (m[_act(i, m)], 0, 0)),
            pl.BlockSpec((1, 1, HE), lambda i, m: (m[_act(i, m)], 0, 0)),
            pl.BlockSpec((1, 1, HE), lambda i, m: (m[_act(i, m)], 0, 0)),
            pl.BlockSpec((1, 1, D), lambda i, m: (m[_act(i, m)], 0, 0)),
        ],
        out_specs=pl.BlockSpec((BT, D), lambda i, m: (_act(i, m), 0)),
    )
    return pl.pallas_call(
        _grouped_body,
        grid_spec=grid_spec,
        out_shape=jax.ShapeDtypeStruct((PMAX, D), _f32),
        compiler_params=pltpu.CompilerParams(
            dimension_semantics=("arbitrary",)),
    )(meta, xs, wg, wu, wd, bg, bu, bd)


def _combine_body(x_ref, z_ref, w3_ref, wrg_ref, wru_ref, wrd_ref,
                  brg_ref, bru_ref, brd_ref, out_ref):
    zb = z_ref[...]
    w1 = w3_ref[0, 0]
    w2 = w3_ref[0, 1]
    moe = w1[:, None] * zb[:TB] + w2[:, None] * zb[TB:]
    xt = x_ref[...].astype(_f32)
    g = jnp.dot(xt, wrg_ref[...], preferred_element_type=_f32) + brg_ref[...]
    u = jnp.dot(xt, wru_ref[...], preferred_element_type=_f32) + bru_ref[...]
    h = jax.nn.silu(g) * u
    res = jnp.dot(h, wrd_ref[...], preferred_element_type=_f32) + brd_ref[...]
    out_ref[...] = moe + res


def _combine(x16, z, w3, wrg, wru, wrd, brg, bru, brd):
    return pl.pallas_call(
        _combine_body,
        grid=(NBLK,),
        in_specs=[
            pl.BlockSpec((TB, D), lambda i: (i, 0)),
            pl.BlockSpec((2 * TB, D), lambda i: (i, 0)),
            pl.BlockSpec((1, 2, TB), lambda i: (i, 0, 0)),
            pl.BlockSpec((D, HR), lambda i: (0, 0)),
            pl.BlockSpec((D, HR), lambda i: (0, 0)),
            pl.BlockSpec((HR, D), lambda i: (0, 0)),
            pl.BlockSpec((1, HR), lambda i: (0, 0)),
            pl.BlockSpec((1, HR), lambda i: (0, 0)),
            pl.BlockSpec((1, D), lambda i: (0, 0)),
        ],
        out_specs=pl.BlockSpec((TB, D), lambda i: (i, 0)),
        out_shape=jax.ShapeDtypeStruct((T, D), _f32),
        compiler_params=pltpu.CompilerParams(
            dimension_semantics=("arbitrary",)),
    )(x16, z, w3, wrg, wru, wrd, brg, bru, brd)


def kernel(x, gate_W, W_gate, W_up, W_down, b_gate, b_up, b_down,
           Wr_gate, Wr_up, Wr_down, br_gate, br_up, br_down):
    w3, x16, pos3, meta = _router(x, gate_W)
    pos3d = pos3.reshape(NW, NCH, CHUNK)
    meta = meta.reshape(TB)
    sidx = jnp.arange(SLOTS, dtype=_i32)
    tok3d = ((sidx // (2 * TB)) * TB + sidx % TB).reshape(NW, NCH, CHUNK)

    xs = _sc_permute(x, pos3d, tok3d)
    y = _grouped(meta, xs, W_gate, W_up, W_down,
                 b_gate.reshape(E, 1, HE), b_up.reshape(E, 1, HE),
                 b_down.reshape(E, 1, D))
    z = _sc_gather(y, pos3d)

    return _combine(x16, z, w3, Wr_gate, Wr_up, Wr_down,
                    br_gate.reshape(1, HR), br_up.reshape(1, HR),
                    br_down.reshape(1, D))


# speedup vs baseline: 1.1111x; 1.0031x over previous
import functools

import jax
import jax.numpy as jnp
from jax import lax
from jax.experimental import pallas as pl
from jax.experimental.pallas import tpu as pltpu
from jax.experimental.pallas import tpu_sc as plsc

T = 4096
D = 1024
E = 16
HE = 256
HR = 256
K = 2
TB = 128
NBLK = T // TB
SLOTS = T * K
BT = 128
PMAX = SLOTS + E * BT
NTILES = PMAX // BT
NW = 32
SPW = SLOTS // NW
CHUNK = 32
NCH = SPW // CHUNK

_f32 = jnp.float32
_i32 = jnp.int32
_bf16 = jnp.bfloat16


def _router_body(x_ref, gw_ref, w3_ref, x16_ref, pos3_ref, meta_ref,
                 carry, e_all, r_all, triu):
    i = pl.program_id(0)

    @pl.when(i == 0)
    def _():
        carry[...] = jnp.zeros_like(carry)
        t0 = lax.broadcasted_iota(_i32, (2 * TB, 2 * TB), 0)
        t1 = lax.broadcasted_iota(_i32, (2 * TB, 2 * TB), 1)
        triu[...] = (t0 < t1).astype(_f32)

    @pl.when(i < NBLK)
    def _():
        xb = x_ref[...]
        x16_ref[...] = xb.astype(_bf16)
        logits = jnp.dot(xb, gw_ref[...], preferred_element_type=_f32)
        lt = logits.T
        m = jnp.max(lt, axis=0, keepdims=True)
        p = jnp.exp(lt - m)
        s = jnp.sum(p, axis=0)

        i1 = jnp.argmax(p, axis=0)
        subl = lax.broadcasted_iota(_i32, (E, TB), 0)
        oh1 = subl == i1[None, :]
        v1 = jnp.max(p, axis=0)
        p2 = jnp.where(oh1, -1.0, p)
        i2 = jnp.argmax(p2, axis=0)
        oh2 = subl == i2[None, :]
        v2 = jnp.max(p2, axis=0)

        w3_ref[0, 0] = v1 / s
        w3_ref[0, 1] = v2 / s
        e_slots = jnp.concatenate([i1, i2], axis=0).astype(_i32)
        e_all[pl.ds(i, 1), :] = e_slots.reshape(1, 2 * TB)

        ohT = jnp.concatenate([oh1, oh2], axis=1).astype(_f32)
        local = jnp.dot(ohT, triu[...], preferred_element_type=_f32)
        rank = jnp.sum((local + carry[...]) * ohT, axis=0)
        r_all[pl.ds(i, 1), :] = rank.reshape(1, 2 * TB)

        carry[...] = carry[...] + jnp.sum(ohT, axis=1, keepdims=True)

    @pl.when(i == NBLK)
    def _():
        c = carry[...]
        pc = jnp.ceil(c / BT) * BT
        r0 = lax.broadcasted_iota(_i32, (E, E), 0)
        r1 = lax.broadcasted_iota(_i32, (E, E), 1)
        tri_l = (r1 < r0).astype(_f32)
        off = jnp.dot(tri_l, pc, preferred_element_type=_f32)

        e = e_all[...]
        acc = jnp.zeros((NBLK, 2 * TB), _f32)
        for ee in range(E):
            acc = acc + jnp.where(e == ee, off[ee:ee + 1, 0:1], 0.0)
        pos3_ref[...] = (acc + r_all[...]).astype(_i32).reshape(NBLK, 1, 2 * TB)

        cum = off + pc
        ts = lax.broadcasted_iota(_i32, (E, TB), 1).astype(_f32) * BT
        te = jnp.sum((ts >= cum).astype(_f32), axis=0)
        te = jnp.minimum(te, float(E - 1))
        total = jnp.sum(pc)
        lane = lax.broadcasted_iota(_i32, (1, TB), 1)
        vec = jnp.where(lane < NTILES, te.reshape(1, TB),
                        jnp.where(lane == NTILES, total, 0.0))
        meta_ref[...] = vec.astype(_i32)


def _router(x, gate_W):
    clamp = NBLK - 1
    return pl.pallas_call(
        _router_body,
        grid=(NBLK + 1,),
        in_specs=[
            pl.BlockSpec((TB, D), lambda i: (jnp.minimum(i, clamp), 0)),
            pl.BlockSpec((D, E), lambda i: (0, 0)),
        ],
        out_specs=[
            pl.BlockSpec((1, 2, TB), lambda i: (jnp.minimum(i, clamp), 0, 0)),
            pl.BlockSpec((TB, D), lambda i: (jnp.minimum(i, clamp), 0)),
            pl.BlockSpec((NBLK, 1, 2 * TB), lambda i: (0, 0, 0)),
            pl.BlockSpec((1, TB), lambda i: (0, 0)),
        ],
        out_shape=[
            jax.ShapeDtypeStruct((NBLK, 2, TB), _f32),
            jax.ShapeDtypeStruct((T, D), _bf16),
            jax.ShapeDtypeStruct((NBLK, 1, 2 * TB), _i32),
            jax.ShapeDtypeStruct((1, TB), _i32),
        ],
        scratch_shapes=[
            pltpu.VMEM((E, 1), _f32),
            pltpu.VMEM((NBLK, 2 * TB), _i32),
            pltpu.VMEM((NBLK, 2 * TB), _f32),
            pltpu.VMEM((2 * TB, 2 * TB), _f32),
        ],
        compiler_params=pltpu.CompilerParams(
            dimension_semantics=("arbitrary",)),
    )(x, gate_W)


def _sc_permute(x, pos3d, tok3d):
    mesh = plsc.VectorSubcoreMesh(core_axis_name="c", subcore_axis_name="s")

    @functools.partial(
        pl.kernel, mesh=mesh,
        out_type=jax.ShapeDtypeStruct((PMAX, D), _f32),
        scratch_types=[
            pltpu.VMEM((NCH, CHUNK), _i32),
            pltpu.VMEM((NCH, CHUNK), _i32),
            pltpu.VMEM((CHUNK, D), _f32),
            pltpu.VMEM((CHUNK, D), _f32),
            pltpu.SemaphoreType.DMA,
            pltpu.SemaphoreType.DMA,
            pltpu.SemaphoreType.DMA,
            pltpu.SemaphoreType.DMA,
        ],
    )
    def k(x_hbm, pos_hbm, tok_hbm, out_hbm,
          tok_v, pos_v, rows0, rows1, g0, g1, s0, s1):
        wid = lax.axis_index("s") * 2 + lax.axis_index("c")
        pltpu.sync_copy(tok_hbm.at[wid], tok_v)
        pltpu.sync_copy(pos_hbm.at[wid], pos_v)
        rows = (rows0, rows1)
        gsem = (g0, g1)
        ssem = (s0, s1)
        gd = [None] * NCH
        sd = [None] * NCH
        gd[0] = pltpu.async_copy(x_hbm.at[tok_v.at[0]], rows[0], gsem[0])
        for ch in range(NCH):
            b = ch % 2
            if ch > 0:
                sd[ch - 1].wait()
            if ch + 1 < NCH:
                gd[ch + 1] = pltpu.async_copy(
                    x_hbm.at[tok_v.at[ch + 1]], rows[1 - b], gsem[1 - b])
            gd[ch].wait()
            sd[ch] = pltpu.async_copy(rows[b], out_hbm.at[pos_v.at[ch]],
                                      ssem[b])
        sd[NCH - 1].wait()

    return k(x, pos3d, tok3d)


def _sc_gather(y, pos3d):
    mesh = plsc.VectorSubcoreMesh(core_axis_name="c", subcore_axis_name="s")

    @functools.partial(
        pl.kernel, mesh=mesh,
        out_type=jax.ShapeDtypeStruct((SLOTS, D), _f32),
        scratch_types=[
            pltpu.VMEM((NCH, CHUNK), _i32),
            pltpu.VMEM((CHUNK, D), _f32),
            pltpu.VMEM((CHUNK, D), _f32),
            pltpu.SemaphoreType.DMA,
            pltpu.SemaphoreType.DMA,
            pltpu.SemaphoreType.DMA,
            pltpu.SemaphoreType.DMA,
        ],
    )
    def k(y_hbm, pos_hbm, out_hbm, pos_v, rows0, rows1, g0, g1, s0, s1):
        wid = lax.axis_index("s") * 2 + lax.axis_index("c")
        base = wid * SPW
        pltpu.sync_copy(pos_hbm.at[wid], pos_v)
        rows = (rows0, rows1)
        gsem = (g0, g1)
        ssem = (s0, s1)
        gd = [None] * NCH
        sd = [None] * NCH
        gd[0] = pltpu.async_copy(y_hbm.at[pos_v.at[0]], rows[0], gsem[0])
        for ch in range(NCH):
            b = ch % 2
            if ch > 0:
                sd[ch - 1].wait()
            if ch + 1 < NCH:
                gd[ch + 1] = pltpu.async_copy(
                    y_hbm.at[pos_v.at[ch + 1]], rows[1 - b], gsem[1 - b])
            gd[ch].wait()
            sd[ch] = pltpu.async_copy(
                rows[b], out_hbm.at[pl.ds(base + ch * CHUNK, CHUNK)], ssem[b])
        sd[NCH - 1].wait()

    return k(y, pos3d)


def _grouped_body(meta_ref, xs_ref, wg_ref, wu_ref, wd_ref,
                  bg_ref, bu_ref, bd_ref, y_ref):
    i = pl.program_id(0)

    @pl.when(i * BT < meta_ref[NTILES])
    def _():
        xt = xs_ref[...]
        g = jnp.dot(xt, wg_ref[0], preferred_element_type=_f32) + bg_ref[0, 0]
        u = jnp.dot(xt, wu_ref[0], preferred_element_type=_f32) + bu_ref[0, 0]
        h = jax.nn.silu(g) * u
        y = jnp.dot(h, wd_ref[0], preferred_element_type=_f32)
        y_ref[...] = y + bd_ref[0, 0]


def _grouped(meta, xs, wg, wu, wd, bg, bu, bd):
    def _act(i, m):
        return jnp.minimum(i, m[NTILES] // BT - 1)

    grid_spec = pltpu.PrefetchScalarGridSpec(
        num_scalar_prefetch=1,
        grid=(NTILES,),
        in_specs=[
            pl.BlockSpec((BT, D), lambda i, m: (_act(i, m), 0)),
            pl.BlockSpec((1, D, HE), lambda i, m: (m[_act(i, m)], 0, 0)),
            pl.BlockSpec((1, D, HE), lambda i, m: (m[_act(i, m)], 0, 0)),
            pl.BlockSpec((1, HE, D), lambda i, m: (m[_act(i, m)], 0, 0)),
            pl.BlockSpec((1, 1, HE), lambda i, m: (m[_act(i, m)], 0, 0)),
            pl.BlockSpec((1, 1, HE), lambda i, m: (m[_act(i, m)], 0, 0)),
            pl.BlockSpec((1, 1, D), lambda i, m: (m[_act(i, m)], 0, 0)),
        ],
        out_specs=pl.BlockSpec((BT, D), lambda i, m: (_act(i, m), 0)),
    )
    return pl.pallas_call(
        _grouped_body,
        grid_spec=grid_spec,
        out_shape=jax.ShapeDtypeStruct((PMAX, D), _f32),
        compiler_params=pltpu.CompilerParams(
            dimension_semantics=("arbitrary",)),
    )(meta, xs, wg, wu, wd, bg, bu, bd)


def _combine_body(x_ref, z_ref, w3_ref, wrg_ref, wru_ref, wrd_ref,
                  brg_ref, bru_ref, brd_ref, out_ref):
    zb = z_ref[...]
    w1 = w3_ref[0, 0]
    w2 = w3_ref[0, 1]
    moe = w1[:, None] * zb[:TB] + w2[:, None] * zb[TB:]
    xt = x_ref[...].astype(_f32)
    g = jnp.dot(xt, wrg_ref[...], preferred_element_type=_f32) + brg_ref[...]
    u = jnp.dot(xt, wru_ref[...], preferred_element_type=_f32) + bru_ref[...]
    h = jax.nn.silu(g) * u
    res = jnp.dot(h, wrd_ref[...], preferred_element_type=_f32) + brd_ref[...]
    out_ref[...] = moe + res


def _combine(x16, z, w3, wrg, wru, wrd, brg, bru, brd):
    return pl.pallas_call(
        _combine_body,
        grid=(NBLK,),
        in_specs=[
            pl.BlockSpec((TB, D), lambda i: (i, 0)),
            pl.BlockSpec((2 * TB, D), lambda i: (i, 0)),
            pl.BlockSpec((1, 2, TB), lambda i: (i, 0, 0)),
            pl.BlockSpec((D, HR), lambda i: (0, 0)),
            pl.BlockSpec((D, HR), lambda i: (0, 0)),
            pl.BlockSpec((HR, D), lambda i: (0, 0)),
            pl.BlockSpec((1, HR), lambda i: (0, 0)),
            pl.BlockSpec((1, HR), lambda i: (0, 0)),
            pl.BlockSpec((1, D), lambda i: (0, 0)),
        ],
        out_specs=pl.BlockSpec((TB, D), lambda i: (i, 0)),
        out_shape=jax.ShapeDtypeStruct((T, D), _f32),
        compiler_params=pltpu.CompilerParams(
            dimension_semantics=("arbitrary",)),
    )(x16, z, w3, wrg, wru, wrd, brg, bru, brd)


def kernel(x, gate_W, W_gate, W_up, W_down, b_gate, b_up, b_down,
           Wr_gate, Wr_up, Wr_down, br_gate, br_up, br_down):
    w3, x16, pos3, meta = _router(x, gate_W)
    pos3d = pos3.reshape(NW, NCH, CHUNK)
    meta = meta.reshape(TB)
    sidx = jnp.arange(SLOTS, dtype=_i32)
    tok3d = ((sidx // (2 * TB)) * TB + sidx % TB).reshape(NW, NCH, CHUNK)

    xs = _sc_permute(x, pos3d, tok3d)
    y = _grouped(meta, xs, W_gate, W_up, W_down,
                 b_gate.reshape(E, 1, HE), b_up.reshape(E, 1, HE),
                 b_down.reshape(E, 1, D))
    z = _sc_gather(y, pos3d)

    return _combine(x16, z, w3, Wr_gate, Wr_up, Wr_down,
                    br_gate.reshape(1, HR), br_up.reshape(1, HR),
                    br_down.reshape(1, D))
